# baseline (device time: 46942 ns/iter reference)
import jax
import jax.numpy as jnp
from jax import lax
from jax.experimental import pallas as pl
from jax.experimental.pallas import tpu as pltpu

N_DEV = 4


def kernel(t, W):
    m, k = t.shape
    k2, n = W.shape

    def body(t_ref, w_ref, out_ref, comm_ref, send_sems, recv_sems):
        my = lax.axis_index("i")
        left = (my - 1) % N_DEV
        right = (my + 1) % N_DEV

        barrier_sem = pltpu.get_barrier_semaphore()
        for nbr in [left, right]:
            pl.semaphore_signal(
                barrier_sem, inc=1,
                device_id=(nbr,), device_id_type=pl.DeviceIdType.MESH,
            )
        pl.semaphore_wait(barrier_sem, 2)

        comm_ref[0, :, :] = t_ref[:, :].astype(jnp.bfloat16)

        for h in range(N_DEV - 1):
            rdma = pltpu.make_async_remote_copy(
                src_ref=comm_ref.at[h],
                dst_ref=comm_ref.at[h + 1],
                send_sem=send_sems.at[h],
                recv_sem=recv_sems.at[h],
                device_id=(right,),
                device_id_type=pl.DeviceIdType.MESH,
            )
            rdma.start()
            rdma.wait()

        s = (
            comm_ref[0, :, :].astype(jnp.float32)
            + comm_ref[1, :, :].astype(jnp.float32)
            + comm_ref[2, :, :].astype(jnp.float32)
            + comm_ref[3, :, :].astype(jnp.float32)
        )
        out_ref[:, :] = jnp.dot(
            s.astype(jnp.bfloat16),
            w_ref[:, :].astype(jnp.bfloat16),
            preferred_element_type=jnp.float32,
        )

    return pl.pallas_call(
        body,
        out_shape=jax.ShapeDtypeStruct((m, n), jnp.float32),
        in_specs=[
            pl.BlockSpec(memory_space=pltpu.VMEM),
            pl.BlockSpec(memory_space=pltpu.VMEM),
        ],
        out_specs=pl.BlockSpec(memory_space=pltpu.VMEM),
        scratch_shapes=[
            pltpu.VMEM((N_DEV, m, k), jnp.bfloat16),
            pltpu.SemaphoreType.DMA((N_DEV - 1,)),
            pltpu.SemaphoreType.DMA((N_DEV - 1,)),
        ],
        compiler_params=pltpu.CompilerParams(collective_id=0),
    )(t, W)


# device time: 23030 ns/iter; 2.0383x vs baseline; 2.0383x over previous
import jax
import jax.numpy as jnp
from jax import lax
from jax.experimental import pallas as pl
from jax.experimental.pallas import tpu as pltpu

N_DEV = 4


def kernel(t, W):
    m, k = t.shape
    k2, n = W.shape
    mc = m // N_DEV

    def body(
        t_ref, w_ref, out_ref,
        stage_ref,
        rs_buf,
        ag_stage,
        ag_buf,
        rs_send_sems, rs_recv_sems, ag_send_sems, ag_recv_sems,
    ):
        my = lax.axis_index("i")

        barrier_sem = pltpu.get_barrier_semaphore()
        for j in range(1, N_DEV):
            pl.semaphore_signal(
                barrier_sem, inc=1,
                device_id=((my + j) % N_DEV,),
                device_id_type=pl.DeviceIdType.MESH,
            )
        pl.semaphore_wait(barrier_sem, N_DEV - 1)

        stage_ref[:, :] = t_ref[:, :].astype(jnp.bfloat16)

        rs_sends = []
        for j in range(1, N_DEV):
            p = (my + j) % N_DEV
            q = N_DEV - 1 - j
            rdma = pltpu.make_async_remote_copy(
                src_ref=stage_ref.at[pl.ds(p * mc, mc), :],
                dst_ref=rs_buf.at[q],
                send_sem=rs_send_sems.at[j - 1],
                recv_sem=rs_recv_sems.at[q],
                device_id=(p,),
                device_id_type=pl.DeviceIdType.MESH,
            )
            rdma.start()
            rs_sends.append(rdma)

        for q in range(N_DEV - 1):
            recv = pltpu.make_async_remote_copy(
                src_ref=rs_buf.at[q],
                dst_ref=rs_buf.at[q],
                send_sem=rs_send_sems.at[0],
                recv_sem=rs_recv_sems.at[q],
                device_id=(my,),
                device_id_type=pl.DeviceIdType.MESH,
            )
            recv.wait_recv()

        s = t_ref[pl.ds(my * mc, mc), :]
        s = s + rs_buf[0].astype(jnp.float32)
        s = s + rs_buf[1].astype(jnp.float32)
        s = s + rs_buf[2].astype(jnp.float32)
        out_chunk = jnp.dot(
            s.astype(jnp.bfloat16),
            w_ref[:, :].astype(jnp.bfloat16),
            preferred_element_type=jnp.float32,
        )
        out_ref[pl.ds(my * mc, mc), :] = out_chunk
        ag_stage[:, :] = out_chunk.astype(jnp.bfloat16)

        ag_sends = []
        for j in range(1, N_DEV):
            p = (my + j) % N_DEV
            q = N_DEV - 1 - j
            rdma = pltpu.make_async_remote_copy(
                src_ref=ag_stage,
                dst_ref=ag_buf.at[q],
                send_sem=ag_send_sems.at[j - 1],
                recv_sem=ag_recv_sems.at[q],
                device_id=(p,),
                device_id_type=pl.DeviceIdType.MESH,
            )
            rdma.start()
            ag_sends.append(rdma)

        for q in range(N_DEV - 1):
            recv = pltpu.make_async_remote_copy(
                src_ref=ag_buf.at[q],
                dst_ref=ag_buf.at[q],
                send_sem=ag_send_sems.at[0],
                recv_sem=ag_recv_sems.at[q],
                device_id=(my,),
                device_id_type=pl.DeviceIdType.MESH,
            )
            recv.wait_recv()
            src = (my + q + 1) % N_DEV
            out_ref[pl.ds(src * mc, mc), :] = ag_buf[q].astype(jnp.float32)

        for rdma in rs_sends + ag_sends:
            rdma.wait_send()

    return pl.pallas_call(
        body,
        out_shape=jax.ShapeDtypeStruct((m, n), jnp.float32),
        in_specs=[
            pl.BlockSpec(memory_space=pltpu.VMEM),
            pl.BlockSpec(memory_space=pltpu.VMEM),
        ],
        out_specs=pl.BlockSpec(memory_space=pltpu.VMEM),
        scratch_shapes=[
            pltpu.VMEM((m, k), jnp.bfloat16),
            pltpu.VMEM((N_DEV - 1, mc, k), jnp.bfloat16),
            pltpu.VMEM((mc, n), jnp.bfloat16),
            pltpu.VMEM((N_DEV - 1, mc, n), jnp.bfloat16),
            pltpu.SemaphoreType.DMA((N_DEV - 1,)),
            pltpu.SemaphoreType.DMA((N_DEV - 1,)),
            pltpu.SemaphoreType.DMA((N_DEV - 1,)),
            pltpu.SemaphoreType.DMA((N_DEV - 1,)),
        ],
        compiler_params=pltpu.CompilerParams(collective_id=0),
    )(t, W)


# device time: 20592 ns/iter; 2.2796x vs baseline; 1.1184x over previous
import jax
import jax.numpy as jnp
from jax import lax
from jax.experimental import pallas as pl
from jax.experimental.pallas import tpu as pltpu

N_DEV = 4
NSUB = 2


def kernel(t, W):
    m, k = t.shape
    k2, n = W.shape
    mc = m // N_DEV
    hc = mc // NSUB

    def body(
        t_ref, w_ref, out_ref,
        stage_ref,
        rs_buf,
        ag_stage,
        ag_buf,
        rs_send_sems, rs_recv_sems, ag_send_sems, ag_recv_sems,
    ):
        my = lax.axis_index("i")

        barrier_sem = pltpu.get_barrier_semaphore()
        for j in range(1, N_DEV):
            pl.semaphore_signal(
                barrier_sem, inc=1,
                device_id=((my + j) % N_DEV,),
                device_id_type=pl.DeviceIdType.MESH,
            )
        pl.semaphore_wait(barrier_sem, N_DEV - 1)

        stage_ref[:, :] = t_ref[:, :].astype(jnp.bfloat16)

        pending = []
        for r in range(NSUB):
            for j in range(1, N_DEV):
                p = (my + j) % N_DEV
                q = N_DEV - 1 - j
                rdma = pltpu.make_async_remote_copy(
                    src_ref=stage_ref.at[pl.ds(p * mc + r * hc, hc), :],
                    dst_ref=rs_buf.at[q, pl.ds(r * hc, hc), :],
                    send_sem=rs_send_sems.at[j - 1, r],
                    recv_sem=rs_recv_sems.at[q, r],
                    device_id=(p,),
                    device_id_type=pl.DeviceIdType.MESH,
                )
                rdma.start()
                pending.append(rdma)

        for r in range(NSUB):
            for q in range(N_DEV - 1):
                recv = pltpu.make_async_remote_copy(
                    src_ref=rs_buf.at[q, pl.ds(r * hc, hc), :],
                    dst_ref=rs_buf.at[q, pl.ds(r * hc, hc), :],
                    send_sem=rs_send_sems.at[0, 0],
                    recv_sem=rs_recv_sems.at[q, r],
                    device_id=(my,),
                    device_id_type=pl.DeviceIdType.MESH,
                )
                recv.wait_recv()

            rows = pl.ds(my * mc + r * hc, hc)
            s = t_ref[rows, :]
            s = s + rs_buf[0, pl.ds(r * hc, hc), :].astype(jnp.float32)
            s = s + rs_buf[1, pl.ds(r * hc, hc), :].astype(jnp.float32)
            s = s + rs_buf[2, pl.ds(r * hc, hc), :].astype(jnp.float32)
            out_half = jnp.dot(
                s.astype(jnp.bfloat16),
                w_ref[:, :].astype(jnp.bfloat16),
                preferred_element_type=jnp.float32,
            )
            out_ref[rows, :] = out_half
            ag_stage[pl.ds(r * hc, hc), :] = out_half.astype(jnp.bfloat16)

            for j in range(1, N_DEV):
                p = (my + j) % N_DEV
                q = N_DEV - 1 - j
                rdma = pltpu.make_async_remote_copy(
                    src_ref=ag_stage.at[pl.ds(r * hc, hc), :],
                    dst_ref=ag_buf.at[q, pl.ds(r * hc, hc), :],
                    send_sem=ag_send_sems.at[j - 1, r],
                    recv_sem=ag_recv_sems.at[q, r],
                    device_id=(p,),
                    device_id_type=pl.DeviceIdType.MESH,
                )
                rdma.start()
                pending.append(rdma)

        for r in range(NSUB):
            for q in range(N_DEV - 1):
                recv = pltpu.make_async_remote_copy(
                    src_ref=ag_buf.at[q, pl.ds(r * hc, hc), :],
                    dst_ref=ag_buf.at[q, pl.ds(r * hc, hc), :],
                    send_sem=ag_send_sems.at[0, 0],
                    recv_sem=ag_recv_sems.at[q, r],
                    device_id=(my,),
                    device_id_type=pl.DeviceIdType.MESH,
                )
                recv.wait_recv()
                src = (my + q + 1) % N_DEV
                out_ref[pl.ds(src * mc + r * hc, hc), :] = (
                    ag_buf[q, pl.ds(r * hc, hc), :].astype(jnp.float32)
                )

        for rdma in pending:
            rdma.wait_send()

    return pl.pallas_call(
        body,
        out_shape=jax.ShapeDtypeStruct((m, n), jnp.float32),
        in_specs=[
            pl.BlockSpec(memory_space=pltpu.VMEM),
            pl.BlockSpec(memory_space=pltpu.VMEM),
        ],
        out_specs=pl.BlockSpec(memory_space=pltpu.VMEM),
        scratch_shapes=[
            pltpu.VMEM((m, k), jnp.bfloat16),
            pltpu.VMEM((N_DEV - 1, mc, k), jnp.bfloat16),
            pltpu.VMEM((mc, n), jnp.bfloat16),
            pltpu.VMEM((N_DEV - 1, mc, n), jnp.bfloat16),
            pltpu.SemaphoreType.DMA((N_DEV - 1, NSUB)),
            pltpu.SemaphoreType.DMA((N_DEV - 1, NSUB)),
            pltpu.SemaphoreType.DMA((N_DEV - 1, NSUB)),
            pltpu.SemaphoreType.DMA((N_DEV - 1, NSUB)),
        ],
        compiler_params=pltpu.CompilerParams(collective_id=0),
    )(t, W)
